# baseline (speedup 1.0000x reference)
import functools

import jax
import jax.numpy as jnp
from jax import lax
from jax.experimental import pallas as pl
from jax.experimental.pallas import tpu as pltpu
from jax.experimental.pallas import tpu_sc as plsc

F32 = jnp.float32
I32 = jnp.int32

_TT = 512
_TM = 256
_LANES = 16


def _top2(bl, E):
    lane = lax.broadcasted_iota(I32, bl.shape, 1)
    m0 = jnp.max(bl, axis=1, keepdims=True)
    a0 = jnp.min(jnp.where(bl == m0, lane, E), axis=1)
    oh0 = (lane == a0[:, None]).astype(F32)
    bl2 = jnp.where(lane == a0[:, None], -jnp.inf, bl)
    m1 = jnp.max(bl2, axis=1, keepdims=True)
    a1 = jnp.min(jnp.where(bl2 == m1, lane, E), axis=1)
    oh1 = (lane == a1[:, None]).astype(F32)
    return oh0, oh1


def _counts_body(x_ref, wr_ref, b_ref, out_ref, xpk_ref):
    j = pl.program_id(0)
    E = wr_ref.shape[0]
    xb = x_ref[...]
    D = xb.shape[1]

    bx = lax.bitcast_convert_type(xb, jnp.uint32)
    r = bx + jnp.uint32(0x7FFF) + ((bx >> jnp.uint32(16)) & jnp.uint32(1))
    h = r >> jnp.uint32(16)
    pk = h[:, :D // 2] | (h[:, D // 2:] << jnp.uint32(16))
    xpk_ref[...] = lax.bitcast_convert_type(pk, I32)

    logits = jnp.dot(xb, wr_ref[...].T, preferred_element_type=F32)
    bl = logits + b_ref[...]
    oh0, oh1 = _top2(bl, E)

    @pl.when(j == 0)
    def _():
        out_ref[...] = jnp.zeros_like(out_ref)

    out_ref[...] += (jnp.sum(oh0, axis=0) + jnp.sum(oh1, axis=0))[None, :]


def _counts_call(flat, Wr, bias2):
    N, D = flat.shape
    E = Wr.shape[0]
    return pl.pallas_call(
        _counts_body,
        grid=(N // _TT,),
        in_specs=[
            pl.BlockSpec((_TT, D), lambda j: (j, 0)),
            pl.BlockSpec((E, D), lambda j: (0, 0)),
            pl.BlockSpec((1, E), lambda j: (0, 0)),
        ],
        out_specs=[
            pl.BlockSpec((1, E), lambda j: (0, 0)),
            pl.BlockSpec((_TT, D // 2), lambda j: (j, 0)),
        ],
        out_shape=[
            jax.ShapeDtypeStruct((1, E), F32),
            jax.ShapeDtypeStruct((N, D // 2), I32),
        ],
    )(flat, Wr, bias2)


def _pos_body(x_ref, wr_ref, b_ref, off_ref, pos0_ref, pos1_ref,
              w0_ref, w1_ref, crun_ref):
    j = pl.program_id(0)
    E = wr_ref.shape[0]
    T = x_ref.shape[0]

    @pl.when(j == 0)
    def _():
        crun_ref[...] = jnp.zeros_like(crun_ref)

    logits = jnp.dot(x_ref[...], wr_ref[...].T, preferred_element_type=F32)
    bl = logits + b_ref[...]
    oh0, oh1 = _top2(bl, E)

    m = jnp.max(logits, axis=1, keepdims=True)
    p = jnp.exp(logits - m)
    probs = p / jnp.sum(p, axis=1, keepdims=True)
    p0 = jnp.sum(probs * oh0, axis=1)
    p1 = jnp.sum(probs * oh1, axis=1)
    s = jnp.clip(p0 + p1, 1e-9, None)
    w0_ref[0, 0, :] = p0 / s
    w1_ref[0, 0, :] = p1 / s

    r = lax.broadcasted_iota(I32, (T, T), 0)
    c = lax.broadcasted_iota(I32, (T, T), 1)
    ltri = (r > c).astype(F32)
    c0 = jnp.dot(ltri, oh0, preferred_element_type=F32)
    c1 = jnp.dot(ltri, oh1, preferred_element_type=F32)
    col0 = jnp.sum(oh0, axis=0)
    col1 = jnp.sum(oh1, axis=0)

    base = off_ref[...] + crun_ref[...]
    pos0 = jnp.sum((base + c0) * oh0, axis=1)
    pos1 = jnp.sum((base + col0[None, :] + c1) * oh1, axis=1)
    pos0_ref[0, 0, :] = pos0.astype(I32)
    pos1_ref[0, 0, :] = pos1.astype(I32)
    crun_ref[...] += (col0 + col1)[None, :]


def _pos_call(flat, Wr, bias2, offs_f):
    N, D = flat.shape
    E = Wr.shape[0]
    NB = N // _TT
    o3 = lambda j: (j, 0, 0)
    return pl.pallas_call(
        _pos_body,
        grid=(NB,),
        in_specs=[
            pl.BlockSpec((_TT, D), lambda j: (j, 0)),
            pl.BlockSpec((E, D), lambda j: (0, 0)),
            pl.BlockSpec((1, E), lambda j: (0, 0)),
            pl.BlockSpec((1, E), lambda j: (0, 0)),
        ],
        out_specs=[
            pl.BlockSpec((1, 1, _TT), o3),
            pl.BlockSpec((1, 1, _TT), o3),
            pl.BlockSpec((1, 1, _TT), o3),
            pl.BlockSpec((1, 1, _TT), o3),
        ],
        out_shape=[
            jax.ShapeDtypeStruct((NB, 1, _TT), I32),
            jax.ShapeDtypeStruct((NB, 1, _TT), I32),
            jax.ShapeDtypeStruct((NB, 1, _TT), F32),
            jax.ShapeDtypeStruct((NB, 1, _TT), F32),
        ],
        scratch_shapes=[pltpu.VMEM((1, E), F32)],
    )(flat, Wr, bias2, offs_f)


def _swiglu_rows(xb, wg, wu, wd):
    dn = (((1,), (1,)), ((), ()))
    g = lax.dot_general(xb, wg, dn, preferred_element_type=F32)
    u = lax.dot_general(xb, wu, dn, preferred_element_type=F32)
    a = g * lax.logistic(g) * u
    return lax.dot_general(a, wd, dn, preferred_element_type=F32)


def _group_body(te_ref, xs_ref, wg_ref, wu_ref, wd_ref, ws_ref, out_ref):
    p = lax.bitcast_convert_type(xs_ref[...], jnp.uint32)
    xlo = lax.bitcast_convert_type(p << jnp.uint32(16), F32)
    xhi = lax.bitcast_convert_type(p & jnp.uint32(0xFFFF0000), F32)
    xb = jnp.concatenate([xlo, xhi], axis=1)
    y = _swiglu_rows(xb, wg_ref[0], wu_ref[0], wd_ref[0])
    out_ref[...] = y * ws_ref[0, 0, :][:, None]


def _group_call(xs, Wg, Wu, Wd, ws3, te):
    PR, DP = xs.shape
    E, HID, D = Wg.shape[0], Wg.shape[1], Wg.shape[2]
    NTJ = PR // _TM
    grid_spec = pltpu.PrefetchScalarGridSpec(
        num_scalar_prefetch=1,
        grid=(NTJ,),
        in_specs=[
            pl.BlockSpec((_TM, DP), lambda j, te: (j, 0)),
            pl.BlockSpec((1, HID, D), lambda j, te: (te[j], 0, 0)),
            pl.BlockSpec((1, HID, D), lambda j, te: (te[j], 0, 0)),
            pl.BlockSpec((1, D, HID), lambda j, te: (te[j], 0, 0)),
            pl.BlockSpec((1, 1, _TM), lambda j, te: (j, 0, 0)),
        ],
        out_specs=pl.BlockSpec((_TM, D), lambda j, te: (j, 0)),
    )
    return pl.pallas_call(
        _group_body,
        grid_spec=grid_spec,
        out_shape=jax.ShapeDtypeStruct((PR, D), F32),
    )(te, xs, Wg, Wu, Wd, ws3)


def _shared_body(x_ref, wg_ref, wu_ref, wd_ref, out_ref):
    out_ref[...] = _swiglu_rows(x_ref[...], wg_ref[...], wu_ref[...],
                                wd_ref[...])


def _shared_call(flat, Wg_s, Wu_s, Wd_s):
    N, D = flat.shape
    HID = Wg_s.shape[0]
    return pl.pallas_call(
        _shared_body,
        grid=(N // _TT,),
        in_specs=[
            pl.BlockSpec((_TT, D), lambda j: (j, 0)),
            pl.BlockSpec((HID, D), lambda j: (0, 0)),
            pl.BlockSpec((HID, D), lambda j: (0, 0)),
            pl.BlockSpec((D, HID), lambda j: (0, 0)),
        ],
        out_specs=pl.BlockSpec((_TT, D), lambda j: (j, 0)),
        out_shape=jax.ShapeDtypeStruct((N, D), F32),
    )(flat, Wg_s, Wu_s, Wd_s)


def _sc_mesh():
    return plsc.VectorSubcoreMesh(core_axis_name="c", subcore_axis_name="s")


def _sc_wid():
    info = plsc.get_sparse_core_info()
    return lax.axis_index("s") * info.num_cores + lax.axis_index("c")


def _dispatch_call(p0, p1, w0, w1, PR):
    N = p0.shape[0]

    def body(p0_hbm, p1_hbm, w0_hbm, w1_hbm, ts_hbm, ws_hbm,
             p0v, p1v, w0v, w1v, tsv, wsv):
        wid = _sc_wid()

        @pl.when(wid == 0)
        def _():
            pltpu.sync_copy(p0_hbm, p0v)
            pltpu.sync_copy(p1_hbm, p1v)

            def zbody(i, carry):
                tsv[pl.ds(i * _LANES, _LANES)] = jnp.zeros((_LANES,), I32)
                return carry

            lax.fori_loop(0, PR // _LANES, zbody, 0)

            def sbody(c, carry):
                sl = pl.ds(c * _LANES, _LANES)
                vals = lax.iota(I32, _LANES) + c * _LANES
                plsc.store_scatter(tsv, [p0v[sl]], vals)
                plsc.store_scatter(tsv, [p1v[sl]], vals)
                return carry

            lax.fori_loop(0, N // _LANES, sbody, 0)
            pltpu.sync_copy(tsv, ts_hbm)

        @pl.when(wid == 1)
        def _():
            pltpu.sync_copy(p0_hbm, p0v)
            pltpu.sync_copy(p1_hbm, p1v)
            pltpu.sync_copy(w0_hbm, w0v)
            pltpu.sync_copy(w1_hbm, w1v)

            def zbody(i, carry):
                wsv[pl.ds(i * _LANES, _LANES)] = jnp.zeros((_LANES,), F32)
                return carry

            lax.fori_loop(0, PR // _LANES, zbody, 0)

            def sbody(c, carry):
                sl = pl.ds(c * _LANES, _LANES)
                plsc.store_scatter(wsv, [p0v[sl]], w0v[sl])
                plsc.store_scatter(wsv, [p1v[sl]], w1v[sl])
                return carry

            lax.fori_loop(0, N // _LANES, sbody, 0)
            pltpu.sync_copy(wsv, ws_hbm)

    k = pl.kernel(
        body,
        out_type=[jax.ShapeDtypeStruct((PR,), I32),
                  jax.ShapeDtypeStruct((PR,), F32)],
        mesh=_sc_mesh(),
        compiler_params=pltpu.CompilerParams(needs_layout_passes=False),
        scratch_types=[
            pltpu.VMEM((N,), I32), pltpu.VMEM((N,), I32),
            pltpu.VMEM((N,), F32), pltpu.VMEM((N,), F32),
            pltpu.VMEM((PR,), I32), pltpu.VMEM((PR,), F32),
        ],
    )
    return k(p0, p1, w0, w1)


def _gather_call(ts, flat, PR):
    N, D = flat.shape
    dt = flat.dtype
    NW = 32
    per_w = PR // NW
    CH = 80
    n_ch = per_w // CH
    NB = 3

    def body(ts_hbm, flat_hbm, xs_hbm, idxv, r0, r1, r2,
             g0, g1, g2, s0, s1, s2):
        wid = _sc_wid()
        base = wid * per_w
        pltpu.sync_copy(ts_hbm.at[pl.ds(base, per_w)], idxv)
        bufs, gsems, ssems = (r0, r1, r2), (g0, g1, g2), (s0, s1, s2)

        def gather(c):
            cp = pltpu.make_async_copy(
                flat_hbm.at[idxv.at[pl.ds(c * CH, CH)]],
                bufs[c % NB], gsems[c % NB])
            cp.start()
            return cp

        def store(c):
            cp = pltpu.make_async_copy(
                bufs[c % NB], xs_hbm.at[pl.ds(base + c * CH, CH)],
                ssems[c % NB])
            cp.start()
            return cp

        g = [None] * n_ch
        s = [None] * n_ch
        for c in range(min(NB, n_ch)):
            g[c] = gather(c)
        for c in range(n_ch):
            g[c].wait()
            s[c] = store(c)
            if c + NB < n_ch:
                s[c].wait()
                g[c + NB] = gather(c + NB)
        for c in range(max(0, n_ch - NB), n_ch):
            if s[c] is not None and c + NB >= n_ch:
                s[c].wait()

    k = pl.kernel(
        body,
        out_type=jax.ShapeDtypeStruct((PR, D), dt),
        mesh=_sc_mesh(),
        scratch_types=[
            pltpu.VMEM((per_w,), I32),
            pltpu.VMEM((CH, D), dt),
            pltpu.VMEM((CH, D), dt),
            pltpu.VMEM((CH, D), dt),
            pltpu.SemaphoreType.DMA, pltpu.SemaphoreType.DMA,
            pltpu.SemaphoreType.DMA, pltpu.SemaphoreType.DMA,
            pltpu.SemaphoreType.DMA, pltpu.SemaphoreType.DMA,
        ],
    )
    return k(ts, flat)


def _combine_call(p0, p1, ys, sh):
    N, D = sh.shape
    NW = 32
    per_w = N // NW
    CH = 16
    n_ch = per_w // CH

    def body(p0_hbm, p1_hbm, ys_hbm, sh_hbm, out_hbm,
             i0v, i1v, b0a, b0b, b1a, b1b, b2a, b2b,
             gsa, gsb, ssa, ssb):
        wid = _sc_wid()
        base = wid * per_w
        pltpu.sync_copy(p0_hbm.at[pl.ds(base, per_w)], i0v)
        pltpu.sync_copy(p1_hbm.at[pl.ds(base, per_w)], i1v)
        b0s, b1s, b2s = (b0a, b0b), (b1a, b1b), (b2a, b2b)
        gsems, ssems = (gsa, gsb), (ssa, ssb)

        def fetch(c):
            m = c % 2
            sl = pl.ds(c * CH, CH)
            cps = [
                pltpu.make_async_copy(ys_hbm.at[i0v.at[sl]], b0s[m], gsems[m]),
                pltpu.make_async_copy(ys_hbm.at[i1v.at[sl]], b1s[m], gsems[m]),
                pltpu.make_async_copy(sh_hbm.at[pl.ds(base + c * CH, CH)],
                                      b2s[m], gsems[m]),
            ]
            for cp in cps:
                cp.start()
            return cps

        def store(c):
            m = c % 2
            cp = pltpu.make_async_copy(
                b0s[m], out_hbm.at[pl.ds(base + c * CH, CH)], ssems[m])
            cp.start()
            return cp

        g = [None] * n_ch
        s = [None] * n_ch
        g[0] = fetch(0)
        for c in range(n_ch):
            m = c % 2
            for cp in g[c]:
                cp.wait()
            if c + 1 < n_ch:
                if c - 1 >= 0:
                    s[c - 1].wait()
                g[c + 1] = fetch(c + 1)

            b0, b1, b2 = b0s[m], b1s[m], b2s[m]

            def rbody(r, carry):
                for cc in range(D // _LANES):
                    sl = pl.ds(cc * _LANES, _LANES)
                    b0[r, sl] = b0[r, sl] + b1[r, sl] + b2[r, sl]
                return carry

            lax.fori_loop(0, CH, rbody, 0)
            s[c] = store(c)
        for c in (n_ch - 2, n_ch - 1):
            if c >= 0:
                s[c].wait()

    k = pl.kernel(
        body,
        out_type=jax.ShapeDtypeStruct((N, D), F32),
        mesh=_sc_mesh(),
        scratch_types=[
            pltpu.VMEM((per_w,), I32), pltpu.VMEM((per_w,), I32),
            pltpu.VMEM((CH, D), F32), pltpu.VMEM((CH, D), F32),
            pltpu.VMEM((CH, D), F32), pltpu.VMEM((CH, D), F32),
            pltpu.VMEM((CH, D), F32), pltpu.VMEM((CH, D), F32),
            pltpu.SemaphoreType.DMA, pltpu.SemaphoreType.DMA,
            pltpu.SemaphoreType.DMA, pltpu.SemaphoreType.DMA,
        ],
    )
    return k(p0, p1, ys, sh)


def kernel(x, W_router, router_bias, Wg, Wu, Wd, Wg_s, Wu_s, Wd_s):
    Bx, Tx, D = x.shape
    N = Bx * Tx
    E = W_router.shape[0]
    K = 2
    flat = x.reshape(N, D).astype(F32)
    bias2 = router_bias.reshape(1, E).astype(F32)

    PR = N * K + E * _TM
    NTJ = PR // _TM

    counts, flat_pk = _counts_call(flat, W_router, bias2)
    expert_load = counts.reshape(E)

    ci = counts.reshape(E).astype(I32)
    caps = ((ci + _TM - 1) // _TM) * _TM
    ends = jnp.cumsum(caps)
    offs = ends - caps
    offs_f = offs.astype(F32).reshape(1, E)
    tile_starts = jnp.arange(NTJ, dtype=I32) * _TM
    te = jnp.clip(
        jnp.sum((tile_starts[:, None] >= ends[None, :]).astype(I32), axis=1),
        0, E - 1).astype(I32)

    pos0, pos1, w0, w1 = _pos_call(flat, W_router, bias2, offs_f)
    p0 = pos0.reshape(N)
    p1 = pos1.reshape(N)
    w0 = w0.reshape(N)
    w1 = w1.reshape(N)

    sh = _shared_call(flat, Wg_s, Wu_s, Wd_s)
    ts, ws = _dispatch_call(p0, p1, w0, w1, PR)
    xs = _gather_call(ts, flat_pk, PR)
    ys = _group_call(xs, Wg, Wu, Wd, ws.reshape(NTJ, 1, _TM), te)
    out = _combine_call(p0, p1, ys, sh)
    return out.reshape(Bx, Tx, D), expert_load


# speedup vs baseline: 1.0018x; 1.0018x over previous
import functools

import jax
import jax.numpy as jnp
from jax import lax
from jax.experimental import pallas as pl
from jax.experimental.pallas import tpu as pltpu
from jax.experimental.pallas import tpu_sc as plsc

F32 = jnp.float32
I32 = jnp.int32

_TT = 512
_TM = 256
_LANES = 16


def _top2(bl, E):
    lane = lax.broadcasted_iota(I32, bl.shape, 1)
    m0 = jnp.max(bl, axis=1, keepdims=True)
    a0 = jnp.min(jnp.where(bl == m0, lane, E), axis=1)
    oh0 = (lane == a0[:, None]).astype(F32)
    bl2 = jnp.where(lane == a0[:, None], -jnp.inf, bl)
    m1 = jnp.max(bl2, axis=1, keepdims=True)
    a1 = jnp.min(jnp.where(bl2 == m1, lane, E), axis=1)
    oh1 = (lane == a1[:, None]).astype(F32)
    return oh0, oh1


def _counts_body(x_ref, wr_ref, b_ref, out_ref, xpk_ref):
    j = pl.program_id(0)
    E = wr_ref.shape[0]
    xb = x_ref[...]
    D = xb.shape[1]

    bx = lax.bitcast_convert_type(xb, jnp.uint32)
    r = bx + jnp.uint32(0x7FFF) + ((bx >> jnp.uint32(16)) & jnp.uint32(1))
    h = r >> jnp.uint32(16)
    pk = h[:, :D // 2] | (h[:, D // 2:] << jnp.uint32(16))
    xpk_ref[...] = lax.bitcast_convert_type(pk, I32)

    logits = jnp.dot(xb, wr_ref[...].T, preferred_element_type=F32)
    bl = logits + b_ref[...]
    oh0, oh1 = _top2(bl, E)

    @pl.when(j == 0)
    def _():
        out_ref[...] = jnp.zeros_like(out_ref)

    out_ref[...] += (jnp.sum(oh0, axis=0) + jnp.sum(oh1, axis=0))[None, :]


def _counts_call(flat, Wr, bias2):
    N, D = flat.shape
    E = Wr.shape[0]
    return pl.pallas_call(
        _counts_body,
        grid=(N // _TT,),
        in_specs=[
            pl.BlockSpec((_TT, D), lambda j: (j, 0)),
            pl.BlockSpec((E, D), lambda j: (0, 0)),
            pl.BlockSpec((1, E), lambda j: (0, 0)),
        ],
        out_specs=[
            pl.BlockSpec((1, E), lambda j: (0, 0)),
            pl.BlockSpec((_TT, D // 2), lambda j: (j, 0)),
        ],
        out_shape=[
            jax.ShapeDtypeStruct((1, E), F32),
            jax.ShapeDtypeStruct((N, D // 2), I32),
        ],
    )(flat, Wr, bias2)


def _pos_body(x_ref, wr_ref, b_ref, off_ref, pos0_ref, pos1_ref,
              w0_ref, w1_ref, crun_ref):
    j = pl.program_id(0)
    E = wr_ref.shape[0]
    T = x_ref.shape[0]

    @pl.when(j == 0)
    def _():
        crun_ref[...] = jnp.zeros_like(crun_ref)

    logits = jnp.dot(x_ref[...], wr_ref[...].T, preferred_element_type=F32)
    bl = logits + b_ref[...]
    oh0, oh1 = _top2(bl, E)

    m = jnp.max(logits, axis=1, keepdims=True)
    p = jnp.exp(logits - m)
    probs = p / jnp.sum(p, axis=1, keepdims=True)
    p0 = jnp.sum(probs * oh0, axis=1)
    p1 = jnp.sum(probs * oh1, axis=1)
    s = jnp.clip(p0 + p1, 1e-9, None)
    w0_ref[0, 0, :] = p0 / s
    w1_ref[0, 0, :] = p1 / s

    r = lax.broadcasted_iota(I32, (T, T), 0)
    c = lax.broadcasted_iota(I32, (T, T), 1)
    ltri = (r > c).astype(F32)
    c0 = jnp.dot(ltri, oh0, preferred_element_type=F32)
    c1 = jnp.dot(ltri, oh1, preferred_element_type=F32)
    col0 = jnp.sum(oh0, axis=0)
    col1 = jnp.sum(oh1, axis=0)

    base = off_ref[...] + crun_ref[...]
    pos0 = jnp.sum((base + c0) * oh0, axis=1)
    pos1 = jnp.sum((base + col0[None, :] + c1) * oh1, axis=1)
    pos0_ref[0, 0, :] = pos0.astype(I32)
    pos1_ref[0, 0, :] = pos1.astype(I32)
    crun_ref[...] += (col0 + col1)[None, :]


def _pos_call(flat, Wr, bias2, offs_f):
    N, D = flat.shape
    E = Wr.shape[0]
    NB = N // _TT
    o3 = lambda j: (j, 0, 0)
    return pl.pallas_call(
        _pos_body,
        grid=(NB,),
        in_specs=[
            pl.BlockSpec((_TT, D), lambda j: (j, 0)),
            pl.BlockSpec((E, D), lambda j: (0, 0)),
            pl.BlockSpec((1, E), lambda j: (0, 0)),
            pl.BlockSpec((1, E), lambda j: (0, 0)),
        ],
        out_specs=[
            pl.BlockSpec((1, 1, _TT), o3),
            pl.BlockSpec((1, 1, _TT), o3),
            pl.BlockSpec((1, 1, _TT), o3),
            pl.BlockSpec((1, 1, _TT), o3),
        ],
        out_shape=[
            jax.ShapeDtypeStruct((NB, 1, _TT), I32),
            jax.ShapeDtypeStruct((NB, 1, _TT), I32),
            jax.ShapeDtypeStruct((NB, 1, _TT), F32),
            jax.ShapeDtypeStruct((NB, 1, _TT), F32),
        ],
        scratch_shapes=[pltpu.VMEM((1, E), F32)],
    )(flat, Wr, bias2, offs_f)


def _swiglu_rows(xb, wg, wu, wd):
    dn = (((1,), (1,)), ((), ()))
    g = lax.dot_general(xb, wg, dn, preferred_element_type=F32)
    u = lax.dot_general(xb, wu, dn, preferred_element_type=F32)
    a = g * lax.logistic(g) * u
    return lax.dot_general(a, wd, dn, preferred_element_type=F32)


def _group_body(te_ref, xs_ref, wg_ref, wu_ref, wd_ref, ws_ref, out_ref):
    p = lax.bitcast_convert_type(xs_ref[...], jnp.uint32)
    xlo = lax.bitcast_convert_type(p << jnp.uint32(16), F32)
    xhi = lax.bitcast_convert_type(p & jnp.uint32(0xFFFF0000), F32)
    xb = jnp.concatenate([xlo, xhi], axis=1)
    y = _swiglu_rows(xb, wg_ref[0], wu_ref[0], wd_ref[0])
    out_ref[...] = y * ws_ref[0, 0, :][:, None]


def _group_call(xs, Wg, Wu, Wd, ws3, te):
    PR, DP = xs.shape
    E, HID, D = Wg.shape[0], Wg.shape[1], Wg.shape[2]
    NTJ = PR // _TM
    grid_spec = pltpu.PrefetchScalarGridSpec(
        num_scalar_prefetch=1,
        grid=(NTJ,),
        in_specs=[
            pl.BlockSpec((_TM, DP), lambda j, te: (j, 0)),
            pl.BlockSpec((1, HID, D), lambda j, te: (te[j], 0, 0)),
            pl.BlockSpec((1, HID, D), lambda j, te: (te[j], 0, 0)),
            pl.BlockSpec((1, D, HID), lambda j, te: (te[j], 0, 0)),
            pl.BlockSpec((1, 1, _TM), lambda j, te: (j, 0, 0)),
        ],
        out_specs=pl.BlockSpec((_TM, D), lambda j, te: (j, 0)),
    )
    return pl.pallas_call(
        _group_body,
        grid_spec=grid_spec,
        out_shape=jax.ShapeDtypeStruct((PR, D), F32),
    )(te, xs, Wg, Wu, Wd, ws3)


def _shared_body(x_ref, wg_ref, wu_ref, wd_ref, out_ref):
    out_ref[...] = _swiglu_rows(x_ref[...], wg_ref[...], wu_ref[...],
                                wd_ref[...])


def _shared_call(flat, Wg_s, Wu_s, Wd_s):
    N, D = flat.shape
    HID = Wg_s.shape[0]
    return pl.pallas_call(
        _shared_body,
        grid=(N // _TT,),
        in_specs=[
            pl.BlockSpec((_TT, D), lambda j: (j, 0)),
            pl.BlockSpec((HID, D), lambda j: (0, 0)),
            pl.BlockSpec((HID, D), lambda j: (0, 0)),
            pl.BlockSpec((D, HID), lambda j: (0, 0)),
        ],
        out_specs=pl.BlockSpec((_TT, D), lambda j: (j, 0)),
        out_shape=jax.ShapeDtypeStruct((N, D), F32),
    )(flat, Wg_s, Wu_s, Wd_s)


def _sc_mesh():
    return plsc.VectorSubcoreMesh(core_axis_name="c", subcore_axis_name="s")


def _sc_wid():
    info = plsc.get_sparse_core_info()
    return lax.axis_index("s") * info.num_cores + lax.axis_index("c")


def _dispatch_call(p0, p1, w0, w1, PR):
    N = p0.shape[0]

    def body(p0_hbm, p1_hbm, w0_hbm, w1_hbm, ts_hbm, ws_hbm,
             p0v, p1v, w0v, w1v, tsv, wsv):
        wid = _sc_wid()

        @pl.when(wid == 0)
        def _():
            pltpu.sync_copy(p0_hbm, p0v)
            pltpu.sync_copy(p1_hbm, p1v)

            def zbody(i, carry):
                tsv[pl.ds(i * _LANES, _LANES)] = jnp.zeros((_LANES,), I32)
                return carry

            lax.fori_loop(0, PR // _LANES, zbody, 0)

            def sbody(c, carry):
                sl = pl.ds(c * _LANES, _LANES)
                vals = lax.iota(I32, _LANES) + c * _LANES
                plsc.store_scatter(tsv, [p0v[sl]], vals)
                plsc.store_scatter(tsv, [p1v[sl]], vals)
                return carry

            lax.fori_loop(0, N // _LANES, sbody, 0)
            pltpu.sync_copy(tsv, ts_hbm)

        @pl.when(wid == 1)
        def _():
            pltpu.sync_copy(p0_hbm, p0v)
            pltpu.sync_copy(p1_hbm, p1v)
            pltpu.sync_copy(w0_hbm, w0v)
            pltpu.sync_copy(w1_hbm, w1v)

            def zbody(i, carry):
                wsv[pl.ds(i * _LANES, _LANES)] = jnp.zeros((_LANES,), F32)
                return carry

            lax.fori_loop(0, PR // _LANES, zbody, 0)

            def sbody(c, carry):
                sl = pl.ds(c * _LANES, _LANES)
                plsc.store_scatter(wsv, [p0v[sl]], w0v[sl])
                plsc.store_scatter(wsv, [p1v[sl]], w1v[sl])
                return carry

            lax.fori_loop(0, N // _LANES, sbody, 0)
            pltpu.sync_copy(wsv, ws_hbm)

    k = pl.kernel(
        body,
        out_type=[jax.ShapeDtypeStruct((PR,), I32),
                  jax.ShapeDtypeStruct((PR,), F32)],
        mesh=_sc_mesh(),
        compiler_params=pltpu.CompilerParams(needs_layout_passes=False),
        scratch_types=[
            pltpu.VMEM((N,), I32), pltpu.VMEM((N,), I32),
            pltpu.VMEM((N,), F32), pltpu.VMEM((N,), F32),
            pltpu.VMEM((PR,), I32), pltpu.VMEM((PR,), F32),
        ],
    )
    return k(p0, p1, w0, w1)


def _gather_call(ts, flat, PR):
    N, D = flat.shape
    dt = flat.dtype
    NW = 32
    per_w = PR // NW
    CH = 40
    n_ch = per_w // CH
    NB = 4

    def body(ts_hbm, flat_hbm, xs_hbm, idxv, r0, r1, r2, r3,
             g0, g1, g2, g3, s0, s1, s2, s3):
        wid = _sc_wid()
        base = wid * per_w
        pltpu.sync_copy(ts_hbm.at[pl.ds(base, per_w)], idxv)
        bufs, gsems, ssems = (r0, r1, r2, r3), (g0, g1, g2, g3), (s0, s1, s2, s3)

        def gather(c):
            cp = pltpu.make_async_copy(
                flat_hbm.at[idxv.at[pl.ds(c * CH, CH)]],
                bufs[c % NB], gsems[c % NB])
            cp.start()
            return cp

        def store(c):
            cp = pltpu.make_async_copy(
                bufs[c % NB], xs_hbm.at[pl.ds(base + c * CH, CH)],
                ssems[c % NB])
            cp.start()
            return cp

        g = [None] * n_ch
        s = [None] * n_ch
        for c in range(min(NB, n_ch)):
            g[c] = gather(c)
        for c in range(n_ch):
            g[c].wait()
            s[c] = store(c)
            if c + NB < n_ch:
                s[c].wait()
                g[c + NB] = gather(c + NB)
        for c in range(max(0, n_ch - NB), n_ch):
            if s[c] is not None and c + NB >= n_ch:
                s[c].wait()

    k = pl.kernel(
        body,
        out_type=jax.ShapeDtypeStruct((PR, D), dt),
        mesh=_sc_mesh(),
        scratch_types=[
            pltpu.VMEM((per_w,), I32),
            pltpu.VMEM((CH, D), dt),
            pltpu.VMEM((CH, D), dt),
            pltpu.VMEM((CH, D), dt),
            pltpu.VMEM((CH, D), dt),
            pltpu.SemaphoreType.DMA, pltpu.SemaphoreType.DMA,
            pltpu.SemaphoreType.DMA, pltpu.SemaphoreType.DMA,
            pltpu.SemaphoreType.DMA, pltpu.SemaphoreType.DMA,
            pltpu.SemaphoreType.DMA, pltpu.SemaphoreType.DMA,
        ],
    )
    return k(ts, flat)


def _combine_call(p0, p1, ys, sh):
    N, D = sh.shape
    NW = 32
    per_w = N // NW
    CH = 16
    n_ch = per_w // CH

    def body(p0_hbm, p1_hbm, ys_hbm, sh_hbm, out_hbm,
             i0v, i1v, b0a, b0b, b1a, b1b, b2a, b2b,
             gsa, gsb, ssa, ssb):
        wid = _sc_wid()
        base = wid * per_w
        pltpu.sync_copy(p0_hbm.at[pl.ds(base, per_w)], i0v)
        pltpu.sync_copy(p1_hbm.at[pl.ds(base, per_w)], i1v)
        b0s, b1s, b2s = (b0a, b0b), (b1a, b1b), (b2a, b2b)
        gsems, ssems = (gsa, gsb), (ssa, ssb)

        def fetch(c):
            m = c % 2
            sl = pl.ds(c * CH, CH)
            cps = [
                pltpu.make_async_copy(ys_hbm.at[i0v.at[sl]], b0s[m], gsems[m]),
                pltpu.make_async_copy(ys_hbm.at[i1v.at[sl]], b1s[m], gsems[m]),
                pltpu.make_async_copy(sh_hbm.at[pl.ds(base + c * CH, CH)],
                                      b2s[m], gsems[m]),
            ]
            for cp in cps:
                cp.start()
            return cps

        def store(c):
            m = c % 2
            cp = pltpu.make_async_copy(
                b0s[m], out_hbm.at[pl.ds(base + c * CH, CH)], ssems[m])
            cp.start()
            return cp

        g = [None] * n_ch
        s = [None] * n_ch
        g[0] = fetch(0)
        for c in range(n_ch):
            m = c % 2
            for cp in g[c]:
                cp.wait()
            if c + 1 < n_ch:
                if c - 1 >= 0:
                    s[c - 1].wait()
                g[c + 1] = fetch(c + 1)

            b0, b1, b2 = b0s[m], b1s[m], b2s[m]

            def rbody(r, carry):
                for cc in range(D // _LANES):
                    sl = pl.ds(cc * _LANES, _LANES)
                    b0[r, sl] = b0[r, sl] + b1[r, sl] + b2[r, sl]
                return carry

            lax.fori_loop(0, CH, rbody, 0)
            s[c] = store(c)
        for c in (n_ch - 2, n_ch - 1):
            if c >= 0:
                s[c].wait()

    k = pl.kernel(
        body,
        out_type=jax.ShapeDtypeStruct((N, D), F32),
        mesh=_sc_mesh(),
        scratch_types=[
            pltpu.VMEM((per_w,), I32), pltpu.VMEM((per_w,), I32),
            pltpu.VMEM((CH, D), F32), pltpu.VMEM((CH, D), F32),
            pltpu.VMEM((CH, D), F32), pltpu.VMEM((CH, D), F32),
            pltpu.VMEM((CH, D), F32), pltpu.VMEM((CH, D), F32),
            pltpu.SemaphoreType.DMA, pltpu.SemaphoreType.DMA,
            pltpu.SemaphoreType.DMA, pltpu.SemaphoreType.DMA,
        ],
    )
    return k(p0, p1, ys, sh)


def kernel(x, W_router, router_bias, Wg, Wu, Wd, Wg_s, Wu_s, Wd_s):
    Bx, Tx, D = x.shape
    N = Bx * Tx
    E = W_router.shape[0]
    K = 2
    flat = x.reshape(N, D).astype(F32)
    bias2 = router_bias.reshape(1, E).astype(F32)

    PR = N * K + E * _TM
    NTJ = PR // _TM

    counts, flat_pk = _counts_call(flat, W_router, bias2)
    expert_load = counts.reshape(E)

    ci = counts.reshape(E).astype(I32)
    caps = ((ci + _TM - 1) // _TM) * _TM
    ends = jnp.cumsum(caps)
    offs = ends - caps
    offs_f = offs.astype(F32).reshape(1, E)
    tile_starts = jnp.arange(NTJ, dtype=I32) * _TM
    te = jnp.clip(
        jnp.sum((tile_starts[:, None] >= ends[None, :]).astype(I32), axis=1),
        0, E - 1).astype(I32)

    pos0, pos1, w0, w1 = _pos_call(flat, W_router, bias2, offs_f)
    p0 = pos0.reshape(N)
    p1 = pos1.reshape(N)
    w0 = w0.reshape(N)
    w1 = w1.reshape(N)

    sh = _shared_call(flat, Wg_s, Wu_s, Wd_s)
    ts, ws = _dispatch_call(p0, p1, w0, w1, PR)
    xs = _gather_call(ts, flat_pk, PR)
    ys = _group_call(xs, Wg, Wu, Wd, ws.reshape(NTJ, 1, _TM), te)
    out = _combine_call(p0, p1, ys, sh)
    return out.reshape(Bx, Tx, D), expert_load


# speedup vs baseline: 1.0027x; 1.0009x over previous
import jax
import jax.numpy as jnp
from jax import lax
from jax.experimental import pallas as pl
from jax.experimental.pallas import tpu as pltpu
from jax.experimental.pallas import tpu_sc as plsc

F32 = jnp.float32
I32 = jnp.int32

_TT = 512
_TM = 256
_LANES = 16


def _top2(bl, E):
    lane = lax.broadcasted_iota(I32, bl.shape, 1)
    m0 = jnp.max(bl, axis=1, keepdims=True)
    a0 = jnp.min(jnp.where(bl == m0, lane, E), axis=1)
    oh0 = (lane == a0[:, None]).astype(F32)
    bl2 = jnp.where(lane == a0[:, None], -jnp.inf, bl)
    m1 = jnp.max(bl2, axis=1, keepdims=True)
    a1 = jnp.min(jnp.where(bl2 == m1, lane, E), axis=1)
    oh1 = (lane == a1[:, None]).astype(F32)
    return oh0, oh1


def _counts_body(x_ref, wr_ref, b_ref, out_ref, xpk_ref):
    j = pl.program_id(0)
    E = wr_ref.shape[0]
    xb = x_ref[...]
    D = xb.shape[1]

    bx = lax.bitcast_convert_type(xb, jnp.uint32)
    r = bx + jnp.uint32(0x7FFF) + ((bx >> jnp.uint32(16)) & jnp.uint32(1))
    h = r >> jnp.uint32(16)
    pk = h[:, :D // 2] | (h[:, D // 2:] << jnp.uint32(16))
    xpk_ref[...] = lax.bitcast_convert_type(pk, I32)

    logits = jnp.dot(xb, wr_ref[...].T, preferred_element_type=F32)
    bl = logits + b_ref[...]
    oh0, oh1 = _top2(bl, E)

    @pl.when(j == 0)
    def _():
        out_ref[...] = jnp.zeros_like(out_ref)

    out_ref[...] += (jnp.sum(oh0, axis=0) + jnp.sum(oh1, axis=0))[None, :]


def _counts_call(flat, Wr, bias2):
    N, D = flat.shape
    E = Wr.shape[0]
    return pl.pallas_call(
        _counts_body,
        grid=(N // _TT,),
        in_specs=[
            pl.BlockSpec((_TT, D), lambda j: (j, 0)),
            pl.BlockSpec((E, D), lambda j: (0, 0)),
            pl.BlockSpec((1, E), lambda j: (0, 0)),
        ],
        out_specs=[
            pl.BlockSpec((1, E), lambda j: (0, 0)),
            pl.BlockSpec((_TT, D // 2), lambda j: (j, 0)),
        ],
        out_shape=[
            jax.ShapeDtypeStruct((1, E), F32),
            jax.ShapeDtypeStruct((N, D // 2), I32),
        ],
    )(flat, Wr, bias2)


def _pos_body(x_ref, wr_ref, b_ref, off_ref, pos0_ref, pos1_ref,
              w0_ref, w1_ref, crun_ref):
    j = pl.program_id(0)
    E = wr_ref.shape[0]
    T = x_ref.shape[0]

    @pl.when(j == 0)
    def _():
        crun_ref[...] = jnp.zeros_like(crun_ref)

    logits = jnp.dot(x_ref[...], wr_ref[...].T, preferred_element_type=F32)
    bl = logits + b_ref[...]
    oh0, oh1 = _top2(bl, E)

    m = jnp.max(logits, axis=1, keepdims=True)
    p = jnp.exp(logits - m)
    probs = p / jnp.sum(p, axis=1, keepdims=True)
    p0 = jnp.sum(probs * oh0, axis=1)
    p1 = jnp.sum(probs * oh1, axis=1)
    s = jnp.clip(p0 + p1, 1e-9, None)
    w0_ref[0, 0, :] = p0 / s
    w1_ref[0, 0, :] = p1 / s

    r = lax.broadcasted_iota(I32, (T, T), 0)
    c = lax.broadcasted_iota(I32, (T, T), 1)
    ltri = (r > c).astype(F32)
    c0 = jnp.dot(ltri, oh0, preferred_element_type=F32)
    c1 = jnp.dot(ltri, oh1, preferred_element_type=F32)
    col0 = jnp.sum(oh0, axis=0)
    col1 = jnp.sum(oh1, axis=0)

    base = off_ref[...] + crun_ref[...]
    pos0 = jnp.sum((base + c0) * oh0, axis=1)
    pos1 = jnp.sum((base + col0[None, :] + c1) * oh1, axis=1)
    pos0_ref[0, 0, :] = pos0.astype(I32)
    pos1_ref[0, 0, :] = pos1.astype(I32)
    crun_ref[...] += (col0 + col1)[None, :]


def _pos_call(flat, Wr, bias2, offs_f):
    N, D = flat.shape
    E = Wr.shape[0]
    NB = N // _TT
    o3 = lambda j: (j, 0, 0)
    return pl.pallas_call(
        _pos_body,
        grid=(NB,),
        in_specs=[
            pl.BlockSpec((_TT, D), lambda j: (j, 0)),
            pl.BlockSpec((E, D), lambda j: (0, 0)),
            pl.BlockSpec((1, E), lambda j: (0, 0)),
            pl.BlockSpec((1, E), lambda j: (0, 0)),
        ],
        out_specs=[
            pl.BlockSpec((1, 1, _TT), o3),
            pl.BlockSpec((1, 1, _TT), o3),
            pl.BlockSpec((1, 1, _TT), o3),
            pl.BlockSpec((1, 1, _TT), o3),
        ],
        out_shape=[
            jax.ShapeDtypeStruct((NB, 1, _TT), I32),
            jax.ShapeDtypeStruct((NB, 1, _TT), I32),
            jax.ShapeDtypeStruct((NB, 1, _TT), F32),
            jax.ShapeDtypeStruct((NB, 1, _TT), F32),
        ],
        scratch_shapes=[pltpu.VMEM((1, E), F32)],
    )(flat, Wr, bias2, offs_f)


def _swiglu_rows(xb, wg, wu, wd):
    dn = (((1,), (1,)), ((), ()))
    g = lax.dot_general(xb, wg, dn, preferred_element_type=F32)
    u = lax.dot_general(xb, wu, dn, preferred_element_type=F32)
    a = g * lax.logistic(g) * u
    return lax.dot_general(a, wd, dn, preferred_element_type=F32)


def _group_body(te_ref, xs_ref, wg_ref, wu_ref, wd_ref, ws_ref, out_ref):
    p = lax.bitcast_convert_type(xs_ref[...], jnp.uint32)
    xlo = lax.bitcast_convert_type(p << jnp.uint32(16), F32)
    xhi = lax.bitcast_convert_type(p & jnp.uint32(0xFFFF0000), F32)
    xb = jnp.concatenate([xlo, xhi], axis=1)
    y = _swiglu_rows(xb, wg_ref[0], wu_ref[0], wd_ref[0])
    out_ref[...] = y * ws_ref[0, 0, :][:, None]


def _group_call(xs, Wg, Wu, Wd, ws3, te):
    PR, DP = xs.shape
    E, HID, D = Wg.shape[0], Wg.shape[1], Wg.shape[2]
    NTJ = PR // _TM
    grid_spec = pltpu.PrefetchScalarGridSpec(
        num_scalar_prefetch=1,
        grid=(NTJ,),
        in_specs=[
            pl.BlockSpec((_TM, DP), lambda j, te: (j, 0)),
            pl.BlockSpec((1, HID, D), lambda j, te: (te[j], 0, 0)),
            pl.BlockSpec((1, HID, D), lambda j, te: (te[j], 0, 0)),
            pl.BlockSpec((1, D, HID), lambda j, te: (te[j], 0, 0)),
            pl.BlockSpec((1, 1, _TM), lambda j, te: (j, 0, 0)),
        ],
        out_specs=pl.BlockSpec((_TM, D), lambda j, te: (j, 0)),
    )
    return pl.pallas_call(
        _group_body,
        grid_spec=grid_spec,
        out_shape=jax.ShapeDtypeStruct((PR, D), F32),
    )(te, xs, Wg, Wu, Wd, ws3)


def _shared_body(x_ref, wg_ref, wu_ref, wd_ref, out_ref):
    out_ref[...] = _swiglu_rows(x_ref[...], wg_ref[...], wu_ref[...],
                                wd_ref[...])


def _shared_call(flat, Wg_s, Wu_s, Wd_s):
    N, D = flat.shape
    HID = Wg_s.shape[0]
    return pl.pallas_call(
        _shared_body,
        grid=(N // _TT,),
        in_specs=[
            pl.BlockSpec((_TT, D), lambda j: (j, 0)),
            pl.BlockSpec((HID, D), lambda j: (0, 0)),
            pl.BlockSpec((HID, D), lambda j: (0, 0)),
            pl.BlockSpec((D, HID), lambda j: (0, 0)),
        ],
        out_specs=pl.BlockSpec((_TT, D), lambda j: (j, 0)),
        out_shape=jax.ShapeDtypeStruct((N, D), F32),
    )(flat, Wg_s, Wu_s, Wd_s)


def _sc_mesh():
    return plsc.VectorSubcoreMesh(core_axis_name="c", subcore_axis_name="s")


def _sc_wid():
    info = plsc.get_sparse_core_info()
    return lax.axis_index("s") * info.num_cores + lax.axis_index("c")


def _dispatch_call(p0, p1, w0, w1, PR):
    N = p0.shape[0]

    def body(p0_hbm, p1_hbm, w0_hbm, w1_hbm, ts_hbm, ws_hbm,
             p0v, p1v, w0v, w1v, tsv, wsv):
        wid = _sc_wid()

        @pl.when(wid == 0)
        def _():
            pltpu.sync_copy(p0_hbm, p0v)
            pltpu.sync_copy(p1_hbm, p1v)

            def zbody(i, carry):
                tsv[pl.ds(i * _LANES, _LANES)] = jnp.zeros((_LANES,), I32)
                return carry

            lax.fori_loop(0, PR // _LANES, zbody, 0)

            def sbody(c, carry):
                sl = pl.ds(c * _LANES, _LANES)
                vals = lax.iota(I32, _LANES) + c * _LANES
                plsc.store_scatter(tsv, [p0v[sl]], vals)
                plsc.store_scatter(tsv, [p1v[sl]], vals)
                return carry

            lax.fori_loop(0, N // _LANES, sbody, 0)
            pltpu.sync_copy(tsv, ts_hbm)

        @pl.when(wid == 1)
        def _():
            pltpu.sync_copy(p0_hbm, p0v)
            pltpu.sync_copy(p1_hbm, p1v)
            pltpu.sync_copy(w0_hbm, w0v)
            pltpu.sync_copy(w1_hbm, w1v)

            def zbody(i, carry):
                wsv[pl.ds(i * _LANES, _LANES)] = jnp.zeros((_LANES,), F32)
                return carry

            lax.fori_loop(0, PR // _LANES, zbody, 0)

            def sbody(c, carry):
                sl = pl.ds(c * _LANES, _LANES)
                plsc.store_scatter(wsv, [p0v[sl]], w0v[sl])
                plsc.store_scatter(wsv, [p1v[sl]], w1v[sl])
                return carry

            lax.fori_loop(0, N // _LANES, sbody, 0)
            pltpu.sync_copy(wsv, ws_hbm)

    k = pl.kernel(
        body,
        out_type=[jax.ShapeDtypeStruct((PR,), I32),
                  jax.ShapeDtypeStruct((PR,), F32)],
        mesh=_sc_mesh(),
        compiler_params=pltpu.CompilerParams(needs_layout_passes=False),
        scratch_types=[
            pltpu.VMEM((N,), I32), pltpu.VMEM((N,), I32),
            pltpu.VMEM((N,), F32), pltpu.VMEM((N,), F32),
            pltpu.VMEM((PR,), I32), pltpu.VMEM((PR,), F32),
        ],
    )
    return k(p0, p1, w0, w1)


def _gather_call(ts, flat, PR):
    N, D = flat.shape
    dt = flat.dtype
    NW = 32
    per_w = PR // NW
    CH = 40
    n_ch = per_w // CH
    NB = 4

    def body(ts_hbm, flat_hbm, xs_hbm, idxv, r0, r1, r2, r3,
             g0, g1, g2, g3, s0, s1, s2, s3):
        wid = _sc_wid()
        base = wid * per_w
        pltpu.sync_copy(ts_hbm.at[pl.ds(base, per_w)], idxv)
        bufs, gsems, ssems = (r0, r1, r2, r3), (g0, g1, g2, g3), (s0, s1, s2, s3)

        def gather(c):
            cp = pltpu.make_async_copy(
                flat_hbm.at[idxv.at[pl.ds(c * CH, CH)]],
                bufs[c % NB], gsems[c % NB])
            cp.start()
            return cp

        def store(c):
            cp = pltpu.make_async_copy(
                bufs[c % NB], xs_hbm.at[pl.ds(base + c * CH, CH)],
                ssems[c % NB])
            cp.start()
            return cp

        g = [None] * n_ch
        s = [None] * n_ch
        for c in range(min(NB, n_ch)):
            g[c] = gather(c)
        for c in range(n_ch):
            g[c].wait()
            s[c] = store(c)
            if c + NB < n_ch:
                s[c].wait()
                g[c + NB] = gather(c + NB)
        for c in range(max(0, n_ch - NB), n_ch):
            if s[c] is not None and c + NB >= n_ch:
                s[c].wait()

    k = pl.kernel(
        body,
        out_type=jax.ShapeDtypeStruct((PR, D), dt),
        mesh=_sc_mesh(),
        scratch_types=[
            pltpu.VMEM((per_w,), I32),
            pltpu.VMEM((CH, D), dt),
            pltpu.VMEM((CH, D), dt),
            pltpu.VMEM((CH, D), dt),
            pltpu.VMEM((CH, D), dt),
            pltpu.SemaphoreType.DMA, pltpu.SemaphoreType.DMA,
            pltpu.SemaphoreType.DMA, pltpu.SemaphoreType.DMA,
            pltpu.SemaphoreType.DMA, pltpu.SemaphoreType.DMA,
            pltpu.SemaphoreType.DMA, pltpu.SemaphoreType.DMA,
        ],
    )
    return k(ts, flat)


def _combine_call(p0, p1, ys, sh):
    N, D = sh.shape
    NW = 32
    per_w = N // NW
    CH = 16
    n_ch = per_w // CH

    def body(p0_hbm, p1_hbm, ys_hbm, sh_hbm, out_hbm,
             i0v, i1v, b0a, b0b, b1a, b1b, b2a, b2b,
             gsa, gsb, ssa, ssb):
        wid = _sc_wid()
        base = wid * per_w
        pltpu.sync_copy(p0_hbm.at[pl.ds(base, per_w)], i0v)
        pltpu.sync_copy(p1_hbm.at[pl.ds(base, per_w)], i1v)
        b0s, b1s, b2s = (b0a, b0b), (b1a, b1b), (b2a, b2b)
        gsems, ssems = (gsa, gsb), (ssa, ssb)

        def fetch(c):
            m = c % 2
            sl = pl.ds(c * CH, CH)
            cps = [
                pltpu.make_async_copy(ys_hbm.at[i0v.at[sl]], b0s[m], gsems[m]),
                pltpu.make_async_copy(ys_hbm.at[i1v.at[sl]], b1s[m], gsems[m]),
                pltpu.make_async_copy(sh_hbm.at[pl.ds(base + c * CH, CH)],
                                      b2s[m], gsems[m]),
            ]
            for cp in cps:
                cp.start()
            return cps

        def store(c):
            m = c % 2
            cp = pltpu.make_async_copy(
                b0s[m], out_hbm.at[pl.ds(base + c * CH, CH)], ssems[m])
            cp.start()
            return cp

        g = [None] * n_ch
        s = [None] * n_ch
        g[0] = fetch(0)
        for c in range(n_ch):
            m = c % 2
            for cp in g[c]:
                cp.wait()
            if c + 1 < n_ch:
                if c - 1 >= 0:
                    s[c - 1].wait()
                g[c + 1] = fetch(c + 1)

            b0, b1, b2 = b0s[m], b1s[m], b2s[m]

            def rbody(r, carry):
                for cc in range(D // _LANES):
                    sl = pl.ds(cc * _LANES, _LANES)
                    b0[r, sl] = b0[r, sl] + b1[r, sl] + b2[r, sl]
                return carry

            lax.fori_loop(0, CH, rbody, 0)
            s[c] = store(c)
        for c in (n_ch - 2, n_ch - 1):
            if c >= 0:
                s[c].wait()

    k = pl.kernel(
        body,
        out_type=jax.ShapeDtypeStruct((N, D), F32),
        mesh=_sc_mesh(),
        scratch_types=[
            pltpu.VMEM((per_w,), I32), pltpu.VMEM((per_w,), I32),
            pltpu.VMEM((CH, D), F32), pltpu.VMEM((CH, D), F32),
            pltpu.VMEM((CH, D), F32), pltpu.VMEM((CH, D), F32),
            pltpu.VMEM((CH, D), F32), pltpu.VMEM((CH, D), F32),
            pltpu.SemaphoreType.DMA, pltpu.SemaphoreType.DMA,
            pltpu.SemaphoreType.DMA, pltpu.SemaphoreType.DMA,
        ],
    )
    return k(p0, p1, ys, sh)


def kernel(x, W_router, router_bias, Wg, Wu, Wd, Wg_s, Wu_s, Wd_s):
    Bx, Tx, D = x.shape
    N = Bx * Tx
    E = W_router.shape[0]
    K = 2
    flat = x.reshape(N, D).astype(F32)
    bias2 = router_bias.reshape(1, E).astype(F32)

    PR = N * K + E * _TM
    NTJ = PR // _TM

    counts, flat_pk = _counts_call(flat, W_router, bias2)
    expert_load = counts.reshape(E)

    ci = counts.reshape(E).astype(I32)
    caps = ((ci + _TM - 1) // _TM) * _TM
    ends = jnp.cumsum(caps)
    offs = ends - caps
    offs_f = offs.astype(F32).reshape(1, E)
    tile_starts = jnp.arange(NTJ, dtype=I32) * _TM
    te = jnp.clip(
        jnp.sum((tile_starts[:, None] >= ends[None, :]).astype(I32), axis=1),
        0, E - 1).astype(I32)

    pos0, pos1, w0, w1 = _pos_call(flat, W_router, bias2, offs_f)
    p0 = pos0.reshape(N)
    p1 = pos1.reshape(N)
    w0 = w0.reshape(N)
    w1 = w1.reshape(N)

    sh = _shared_call(flat, Wg_s, Wu_s, Wd_s)
    ts, ws = _dispatch_call(p0, p1, w0, w1, PR)
    xs = _gather_call(ts, flat_pk, PR)
    ys = _group_call(xs, Wg, Wu, Wd, ws.reshape(NTJ, 1, _TM), te)
    out = _combine_call(p0, p1, ys, sh)
    return out.reshape(Bx, Tx, D), expert_load
